# baseline (speedup 1.0000x reference)
import jax
import jax.numpy as jnp
from jax.experimental import pallas as pl
from jax.experimental.pallas import tpu as pltpu

BM = 512
BN = 2048
KQ = 512
NI = 16


def _linear_kernel(x_hbm, w_hbm, b_ref, o_hbm, w_vmem, x_buf, o_buf,
                   x_sem, o_sem, w_sem):
    K = w_vmem.shape[1]
    n_q = K // KQ
    n_steps = 2 * NI

    def _w_cp(j, q):
        return pltpu.make_async_copy(
            w_hbm.at[pl.ds(j * BN, BN), pl.ds(q * KQ, KQ)],
            w_vmem.at[:, pl.ds(q * KQ, KQ)],
            w_sem.at[q])

    def _x_cp(s, slot):
        i = jax.lax.rem(s, NI)
        return pltpu.make_async_copy(
            x_hbm.at[pl.ds(i * BM, BM), :], x_buf.at[slot], x_sem.at[slot])

    def _o_cp(s, slot):
        j = s // NI
        i = jax.lax.rem(s, NI)
        return pltpu.make_async_copy(
            o_buf.at[slot],
            o_hbm.at[pl.ds(i * BM, BM), pl.ds(j * BN, BN)],
            o_sem.at[slot])

    _x_cp(0, 0).start()
    for q in range(n_q):
        _w_cp(0, q).start()

    def _dims():
        return (((1,), (1,)), ((), ()))

    def body(s, _):
        slot = jax.lax.rem(s, 2)
        j = s // NI

        @pl.when(s + 1 < n_steps)
        def _():
            _x_cp(s + 1, 1 - slot).start()

        _x_cp(s, slot).wait()

        @pl.when(s >= 2)
        def _():
            _o_cp(s, slot).wait()

        @pl.when(jax.lax.rem(s, NI) == 0)
        def _switch_step():
            for q in range(n_q):
                _w_cp(j, q).wait()
                part = jax.lax.dot_general(
                    x_buf[slot][:, q * KQ:(q + 1) * KQ],
                    w_vmem[:, q * KQ:(q + 1) * KQ],
                    dimension_numbers=_dims(),
                    preferred_element_type=jnp.float32,
                )
                if q == 0:
                    for c in range(4):
                        cs = c * (BN // 4)
                        o_buf[slot, :, cs:cs + BN // 4] = (
                            part[:, cs:cs + BN // 4]
                            + b_ref[:, pl.ds(j * BN + cs, BN // 4)])
                else:
                    o_buf[slot] += part

        @pl.when(jax.lax.rem(s, NI) != 0)
        def _steady_step():
            acc = jax.lax.dot_general(
                x_buf[slot][...], w_vmem[...],
                dimension_numbers=_dims(),
                preferred_element_type=jnp.float32,
            )
            for c in range(4):
                cs = c * (BN // 4)
                o_buf[slot, :, cs:cs + BN // 4] = (
                    acc[:, cs:cs + BN // 4]
                    + b_ref[:, pl.ds(j * BN + cs, BN // 4)])

        _o_cp(s, slot).start()

        @pl.when((jax.lax.rem(s, NI) == NI - 1) & (j + 1 < 2))
        def _issue_next_half():
            for q in range(n_q):
                _w_cp(j + 1, q).start()

        return 0

    jax.lax.fori_loop(0, n_steps, body, 0)

    _o_cp(n_steps - 2, (n_steps - 2) % 2).wait()
    _o_cp(n_steps - 1, (n_steps - 1) % 2).wait()


def kernel(input, weight, bias):
    M, K = input.shape
    N = weight.shape[0]
    return pl.pallas_call(
        _linear_kernel,
        in_specs=[
            pl.BlockSpec(memory_space=pl.ANY),
            pl.BlockSpec(memory_space=pl.ANY),
            pl.BlockSpec((1, N), lambda: (0, 0)),
        ],
        out_specs=pl.BlockSpec(memory_space=pl.ANY),
        out_shape=jax.ShapeDtypeStruct((M, N), jnp.float32),
        scratch_shapes=[
            pltpu.VMEM((BN, K), jnp.float32),
            pltpu.VMEM((2, BM, K), jnp.float32),
            pltpu.VMEM((2, BM, BN), jnp.float32),
            pltpu.SemaphoreType.DMA((2,)),
            pltpu.SemaphoreType.DMA((2,)),
            pltpu.SemaphoreType.DMA((8,)),
        ],
        compiler_params=pltpu.CompilerParams(
            vmem_limit_bytes=60000 * 1024,
        ),
        name="safe_linear",
    )(input, weight, bias.reshape(1, N))


# speedup vs baseline: 1.0095x; 1.0095x over previous
import jax
import jax.numpy as jnp
from jax.experimental import pallas as pl
from jax.experimental.pallas import tpu as pltpu

BM = 512
BN = 2048
KQ = 512


def _linear_kernel(x_ref, w_hbm, b_ref, o_ref, w_vmem, w_sem):
    j = pl.program_id(0)
    K = x_ref.shape[1]
    n_q = K // KQ

    def _cp(q):
        return pltpu.make_async_copy(
            w_hbm.at[pl.ds(j * BN, BN), pl.ds(q * KQ, KQ)],
            w_vmem.at[:, pl.ds(q * KQ, KQ)],
            w_sem.at[q])

    def _dims():
        return (((1,), (1,)), ((), ()))

    @pl.when(pl.program_id(1) == 0)
    def _transition():
        for q in range(n_q):
            _cp(q).start()
        for q in range(n_q):
            _cp(q).wait()
            part = jax.lax.dot_general(
                x_ref[:, q * KQ:(q + 1) * KQ],
                w_vmem[:, q * KQ:(q + 1) * KQ],
                dimension_numbers=_dims(),
                preferred_element_type=jnp.float32,
            )
            if q == 0:
                o_ref[...] = part + b_ref[...]
            else:
                o_ref[...] += part

    @pl.when(pl.program_id(1) > 0)
    def _steady():
        acc = jax.lax.dot_general(
            x_ref[...], w_vmem[...],
            dimension_numbers=_dims(),
            preferred_element_type=jnp.float32,
        )
        o_ref[...] = acc + b_ref[...]


def kernel(input, weight, bias):
    M, K = input.shape
    N = weight.shape[0]
    grid = (N // BN, M // BM)
    return pl.pallas_call(
        _linear_kernel,
        grid=grid,
        in_specs=[
            pl.BlockSpec((BM, K), lambda j, i: (i, 0)),
            pl.BlockSpec(memory_space=pl.ANY),
            pl.BlockSpec((1, BN), lambda j, i: (0, j)),
        ],
        out_specs=pl.BlockSpec((BM, BN), lambda j, i: (i, j)),
        out_shape=jax.ShapeDtypeStruct((M, N), jnp.float32),
        scratch_shapes=[
            pltpu.VMEM((BN, K), jnp.float32),
            pltpu.SemaphoreType.DMA((8,)),
        ],
        compiler_params=pltpu.CompilerParams(
            dimension_semantics=("arbitrary", "arbitrary"),
            vmem_limit_bytes=60000 * 1024,
        ),
        name="safe_linear",
    )(input, weight, bias.reshape(1, N))
